# Initial kernel scaffold; baseline (speedup 1.0000x reference)
import jax
import jax.numpy as jnp
from jax.experimental import pallas as pl


def kernel(num_features, cat_features, W_num_0, W_num_1, W_num_2, W_num_3, W_num_4, W_num_5, W_num_6, W_num_7, W_cat_0, W_cat_1, W_cat_2, W_cat_3, W_cat_4, W_cat_5, W_cat_6, W_cat_7, W_cat_8):
    raise NotImplementedError("write your pallas kernel here")


# speedup vs baseline: 6.5414x; 6.5414x over previous
import functools

import jax
import jax.numpy as jnp
import numpy as np
from jax import lax
from jax.experimental import pallas as pl
from jax.experimental.pallas import tpu as pltpu
from jax.experimental.pallas import tpu_sc as plsc

_CAT_SIZES = (512, 128, 256, 256, 64, 256, 256, 16, 256)
_NUM_SIZES = (64, 16, 128, 64, 128, 64, 512, 512)
_DIM = 16
_BATCH = 16384
_NF = len(_NUM_SIZES) + len(_CAT_SIZES)

_NC, _NS = 2, 16
_NW = _NC * _NS
_CHUNK = 256
_POS = _CHUNK * _NF
_NCHUNK = _BATCH // _CHUNK
_GROUP = 128
_NGATHER = _POS // _GROUP


def _feature_params():
    sizes = list(_NUM_SIZES) + list(_CAT_SIZES)
    offs = np.cumsum([0] + sizes[:-1]).astype(np.int32)
    scale = np.zeros(16 * _NF, np.float32)
    clipmax = np.zeros(16 * _NF, np.int32)
    andmask = np.zeros(16 * _NF, np.int32)
    iscat = np.zeros(16 * _NF, np.int32)
    offset = np.zeros(16 * _NF, np.int32)
    for q in range(16 * _NF):
        f = q % _NF
        offset[q] = offs[f]
        if f < len(_NUM_SIZES):
            scale[q] = float(_NUM_SIZES[f] - 1)
            clipmax[q] = _NUM_SIZES[f] - 1
        else:
            iscat[q] = 1
            andmask[q] = _CAT_SIZES[f - len(_NUM_SIZES)] - 1
    return scale, clipmax, andmask, iscat, offset


def _encoder_body(comb_hbm, wall_hbm, scale_hbm, clip_hbm, mask_hbm,
                  sel_hbm, off_hbm, out_hbm,
                  comb_v, idx_v, out_v, scale_v, clip_v, mask_v, sel_v,
                  off_v, sem):
    wid = lax.axis_index("s") * _NC + lax.axis_index("c")

    pltpu.sync_copy(scale_hbm, scale_v)
    pltpu.sync_copy(clip_hbm, clip_v)
    pltpu.sync_copy(mask_hbm, mask_v)
    pltpu.sync_copy(sel_hbm, sel_v)
    pltpu.sync_copy(off_hbm, off_v)

    for c in range(_NCHUNK // _NW):
        chunk = wid * (_NCHUNK // _NW) + c
        base = chunk * _POS

        pltpu.sync_copy(comb_hbm.at[pl.ds(base, _POS)], comb_v)

        def superblock(s, _):
            for k in range(_NF):
                pq = k * 16
                x = comb_v[pl.ds(s * (16 * _NF) + pq, 16)]
                xf = lax.bitcast_convert_type(x, jnp.float32)
                ni = (xf * scale_v[pl.ds(pq, 16)]).astype(jnp.int32)
                ni = jnp.minimum(jnp.maximum(ni, 0), clip_v[pl.ds(pq, 16)])
                ci = x & mask_v[pl.ds(pq, 16)]
                idx = jnp.where(sel_v[pl.ds(pq, 16)] != 0, ci, ni)
                idx = idx + off_v[pl.ds(pq, 16)]
                t = s * _NF + k
                idx_v[t // 8, pl.ds((t % 8) * 16, 16)] = idx
            return 0

        lax.fori_loop(0, _CHUNK // 16, superblock, 0)

        def gather(j, _):
            pltpu.async_copy(wall_hbm.at[idx_v.at[j]],
                             out_v.at[pl.ds(j * _GROUP, _GROUP)], sem)
            return 0

        lax.fori_loop(0, _NGATHER, gather, 0)
        pltpu.make_async_copy(out_hbm.at[pl.ds(base, _POS)], out_v, sem).wait()

        pltpu.sync_copy(out_v, out_hbm.at[pl.ds(base, _POS)])


@functools.partial(
    pl.kernel,
    out_type=jax.ShapeDtypeStruct((_BATCH * _NF, _DIM), jnp.float32),
    mesh=plsc.VectorSubcoreMesh(core_axis_name="c", subcore_axis_name="s"),
    scratch_types=[
        pltpu.VMEM((_POS,), jnp.int32),
        pltpu.VMEM((_NGATHER, _GROUP), jnp.int32),
        pltpu.VMEM((_POS, _DIM), jnp.float32),
        pltpu.VMEM((16 * _NF,), jnp.float32),
        pltpu.VMEM((16 * _NF,), jnp.int32),
        pltpu.VMEM((16 * _NF,), jnp.int32),
        pltpu.VMEM((16 * _NF,), jnp.int32),
        pltpu.VMEM((16 * _NF,), jnp.int32),
        pltpu.SemaphoreType.DMA,
    ],
    compiler_params=pltpu.CompilerParams(use_tc_tiling_on_sc=False),
)
def _encoder(*refs):
    _encoder_body(*refs)


def kernel(num_features, cat_features,
           W_num_0, W_num_1, W_num_2, W_num_3, W_num_4, W_num_5, W_num_6,
           W_num_7, W_cat_0, W_cat_1, W_cat_2, W_cat_3, W_cat_4, W_cat_5,
           W_cat_6, W_cat_7, W_cat_8):
    wall = jnp.concatenate([
        W_num_0, W_num_1, W_num_2, W_num_3, W_num_4, W_num_5, W_num_6,
        W_num_7, W_cat_0, W_cat_1, W_cat_2, W_cat_3, W_cat_4, W_cat_5,
        W_cat_6, W_cat_7, W_cat_8], axis=0)
    comb = jnp.concatenate(
        [lax.bitcast_convert_type(num_features, jnp.int32), cat_features],
        axis=1).reshape(-1)
    scale, clipmax, andmask, iscat, offset = _feature_params()
    out = _encoder(comb, wall, jnp.asarray(scale), jnp.asarray(clipmax),
                   jnp.asarray(andmask), jnp.asarray(iscat),
                   jnp.asarray(offset))
    return out.reshape(_BATCH, _NF * _DIM)
